# Initial kernel scaffold; baseline (speedup 1.0000x reference)
import jax
import jax.numpy as jnp
from jax.experimental import pallas as pl


def kernel(atom_feats, bond_feats, edge_index, W1, b1, W2, b2, W3, b3):
    raise NotImplementedError("write your pallas kernel here")


# speedup vs baseline: 1.6888x; 1.6888x over previous
import functools

import jax
import jax.numpy as jnp
from jax import lax
from jax.experimental import pallas as pl
from jax.experimental.pallas import tpu as pltpu
from jax.experimental.pallas import tpu_sc as plsc

N_NODES = 10000
N_EDGES = 320000
D_FEAT = 128
D_EDGE = 16
OUT_DIM = 128

NC = 2
NS = 16
NW = NC * NS
EW = N_EDGES // NW
C = 80
NCHUNK = EW // C

_HIGH = lax.Precision.HIGHEST


def _leaky(x):
    return jnp.where(x >= 0, x, 0.01 * x)


def _node_proj_body(atom_ref, w_ref, out_ref):
    out_ref[...] = jnp.dot(atom_ref[...], w_ref[...],
                           preferred_element_type=jnp.float32,
                           precision=_HIGH)


def _node_proj(atom_feats, w1b):
    return pl.pallas_call(
        _node_proj_body,
        out_shape=jax.ShapeDtypeStruct((N_NODES, OUT_DIM), jnp.float32),
    )(atom_feats, w1b)


@functools.partial(
    pl.kernel,
    mesh=plsc.VectorSubcoreMesh(core_axis_name="c", subcore_axis_name="s"),
    out_type=jax.ShapeDtypeStruct((N_EDGES, OUT_DIM), jnp.float32),
    scratch_types=[
        pltpu.VMEM((EW,), jnp.int32),
        pltpu.VMEM((EW,), jnp.int32),
        pltpu.VMEM((C, OUT_DIM), jnp.float32),
        pltpu.VMEM((C, OUT_DIM), jnp.float32),
        pltpu.SemaphoreType.DMA,
    ],
)
def _gather_add(table, src_hbm, dst_hbm, out_hbm,
                idx_s, idx_d, rows_s, rows_d, sem):
    wid = lax.axis_index("s") * NC + lax.axis_index("c")
    base = wid * EW
    pltpu.sync_copy(src_hbm.at[pl.ds(base, EW)], idx_s)
    pltpu.sync_copy(dst_hbm.at[pl.ds(base, EW)], idx_d)

    def chunk(j, carry):
        off = base + j * C
        c1 = pltpu.async_copy(table.at[idx_s.at[pl.ds(j * C, C)]], rows_s, sem)
        c2 = pltpu.async_copy(table.at[idx_d.at[pl.ds(j * C, C)]], rows_d, sem)
        c1.wait()
        c2.wait()

        def add_row(e, carry2):
            for k in range(OUT_DIM // 16):
                sl = pl.ds(k * 16, 16)
                rows_s[e, sl] = rows_s[e, sl] + rows_d[e, sl]
            return carry2

        lax.fori_loop(0, C, add_row, 0)
        pltpu.sync_copy(rows_s, out_hbm.at[pl.ds(off, C)])
        return carry

    lax.fori_loop(0, NCHUNK, chunk, 0)


BE = 2000


def _mlp_body(bond_ref, s_ref, w1a_ref, b1_ref, w2_ref, b2_ref,
              w3_ref, b3_ref, out_ref):
    h = jnp.dot(bond_ref[...], w1a_ref[...],
                preferred_element_type=jnp.float32, precision=_HIGH)
    h = _leaky(h + s_ref[...] + b1_ref[...])
    h = _leaky(jnp.dot(h, w2_ref[...],
                       preferred_element_type=jnp.float32,
                       precision=_HIGH) + b2_ref[...])
    h = _leaky(jnp.dot(h, w3_ref[...],
                       preferred_element_type=jnp.float32,
                       precision=_HIGH) + b3_ref[...])
    out_ref[...] = h


def _edge_mlp(bond_feats, s, w1a, b1, w2, b2, w3, b3):
    grid = (N_EDGES // BE,)
    full = lambda shape: pl.BlockSpec(shape, lambda i: (0, 0))
    return pl.pallas_call(
        _mlp_body,
        grid=grid,
        in_specs=[
            pl.BlockSpec((BE, D_EDGE), lambda i: (i, 0)),
            pl.BlockSpec((BE, OUT_DIM), lambda i: (i, 0)),
            full((D_EDGE, OUT_DIM)),
            full((1, OUT_DIM)),
            full((OUT_DIM, OUT_DIM)),
            full((1, OUT_DIM)),
            full((OUT_DIM, OUT_DIM)),
            full((1, OUT_DIM)),
        ],
        out_specs=pl.BlockSpec((BE, OUT_DIM), lambda i: (i, 0)),
        out_shape=jax.ShapeDtypeStruct((N_EDGES, OUT_DIM), jnp.float32),
    )(bond_feats, s, w1a, b1, w2, b2, w3, b3)


def kernel(atom_feats, bond_feats, edge_index, W1, b1, W2, b2, W3, b3):
    w1a = W1[:D_EDGE]
    w1b = W1[D_EDGE:]
    src = edge_index[0]
    dst = edge_index[1]
    atom_w = _node_proj(atom_feats, w1b)
    s = _gather_add(atom_w, src, dst)
    return _edge_mlp(bond_feats, s,
                     w1a, b1.reshape(1, -1),
                     W2, b2.reshape(1, -1),
                     W3, b3.reshape(1, -1))


# speedup vs baseline: 2.7750x; 1.6432x over previous
import functools

import jax
import jax.numpy as jnp
from jax import lax
from jax.experimental import pallas as pl
from jax.experimental.pallas import tpu as pltpu
from jax.experimental.pallas import tpu_sc as plsc

N_NODES = 10000
N_EDGES = 320000
D_FEAT = 128
D_EDGE = 16
OUT_DIM = 128

NC = 2
NS = 16
NW = NC * NS
EW = N_EDGES // NW
C = 80
NCHUNK = EW // C

_HIGH = lax.Precision.HIGHEST


def _leaky(x):
    return jnp.where(x >= 0, x, 0.01 * x)


def _node_proj_body(atom_ref, w_ref, out_ref):
    out_ref[...] = jnp.dot(atom_ref[...], w_ref[...],
                           preferred_element_type=jnp.float32,
                           precision=_HIGH)


def _node_proj(atom_feats, w1b):
    return pl.pallas_call(
        _node_proj_body,
        out_shape=jax.ShapeDtypeStruct((N_NODES, OUT_DIM), jnp.float32),
    )(atom_feats, w1b)


@functools.partial(
    pl.kernel,
    mesh=plsc.VectorSubcoreMesh(core_axis_name="c", subcore_axis_name="s"),
    out_type=jax.ShapeDtypeStruct((N_EDGES, OUT_DIM), jnp.float32),
    scratch_types=[
        pltpu.VMEM((EW,), jnp.int32),
        pltpu.VMEM((EW,), jnp.int32),
        pltpu.VMEM((C, OUT_DIM), jnp.float32),
        pltpu.VMEM((C, OUT_DIM), jnp.float32),
        pltpu.SemaphoreType.DMA,
    ],
)
def _gather_add(table, src_hbm, dst_hbm, out_hbm,
                idx_s, idx_d, rows_s, rows_d, sem):
    wid = lax.axis_index("s") * NC + lax.axis_index("c")
    base = wid * EW
    pltpu.sync_copy(src_hbm.at[pl.ds(base, EW)], idx_s)
    pltpu.sync_copy(dst_hbm.at[pl.ds(base, EW)], idx_d)

    def chunk(j, carry):
        off = base + j * C
        c1 = pltpu.async_copy(table.at[idx_s.at[pl.ds(j * C, C)]], rows_s, sem)
        c2 = pltpu.async_copy(table.at[idx_d.at[pl.ds(j * C, C)]], rows_d, sem)
        c1.wait()
        c2.wait()

        def add_row(e, carry2):
            for k in range(OUT_DIM // 16):
                sl = pl.ds(k * 16, 16)
                rows_s[e, sl] = rows_s[e, sl] + rows_d[e, sl]
            return carry2

        lax.fori_loop(0, C, add_row, 0)
        pltpu.sync_copy(rows_s, out_hbm.at[pl.ds(off, C)])
        return carry

    lax.fori_loop(0, NCHUNK, chunk, 0)


BE = 2000


def _mlp_body(bond_ref, s_ref, w1a_ref, b1_ref, w2_ref, b2_ref,
              w3_ref, b3_ref, out_ref):
    h = jnp.dot(bond_ref[...], w1a_ref[...],
                preferred_element_type=jnp.float32)
    h = _leaky(h + s_ref[...] + b1_ref[...])
    h = _leaky(jnp.dot(h.astype(jnp.bfloat16), w2_ref[...],
                       preferred_element_type=jnp.float32) + b2_ref[...])
    h = _leaky(jnp.dot(h.astype(jnp.bfloat16), w3_ref[...],
                       preferred_element_type=jnp.float32) + b3_ref[...])
    out_ref[...] = h


def _edge_mlp(bond_feats, s, w1a, b1, w2, b2, w3, b3):
    grid = (N_EDGES // BE,)
    bond_feats = bond_feats.astype(jnp.bfloat16)
    w1a = w1a.astype(jnp.bfloat16)
    w2 = w2.astype(jnp.bfloat16)
    w3 = w3.astype(jnp.bfloat16)
    full = lambda shape: pl.BlockSpec(shape, lambda i: (0, 0))
    return pl.pallas_call(
        _mlp_body,
        grid=grid,
        in_specs=[
            pl.BlockSpec((BE, D_EDGE), lambda i: (i, 0)),
            pl.BlockSpec((BE, OUT_DIM), lambda i: (i, 0)),
            full((D_EDGE, OUT_DIM)),
            full((1, OUT_DIM)),
            full((OUT_DIM, OUT_DIM)),
            full((1, OUT_DIM)),
            full((OUT_DIM, OUT_DIM)),
            full((1, OUT_DIM)),
        ],
        out_specs=pl.BlockSpec((BE, OUT_DIM), lambda i: (i, 0)),
        out_shape=jax.ShapeDtypeStruct((N_EDGES, OUT_DIM), jnp.float32),
    )(bond_feats, s, w1a, b1, w2, b2, w3, b3)


def kernel(atom_feats, bond_feats, edge_index, W1, b1, W2, b2, W3, b3):
    w1a = W1[:D_EDGE]
    w1b = W1[D_EDGE:]
    src = edge_index[0]
    dst = edge_index[1]
    atom_w = _node_proj(atom_feats, w1b)
    s = _gather_add(atom_w, src, dst)
    return _edge_mlp(bond_feats, s,
                     w1a, b1.reshape(1, -1),
                     W2, b2.reshape(1, -1),
                     W3, b3.reshape(1, -1))


# speedup vs baseline: 3.0495x; 1.0989x over previous
import functools

import jax
import jax.numpy as jnp
import numpy as np
from jax import lax
from jax.experimental import pallas as pl
from jax.experimental.pallas import tpu as pltpu
from jax.experimental.pallas import tpu_sc as plsc

N_NODES = 10000
N_EDGES = 320000
D_FEAT = 128
D_EDGE = 16
OUT_DIM = 128
PK = OUT_DIM // 2

NC = 2
NS = 16
NW = NC * NS
EW = N_EDGES // NW
C = 80
NCHUNK = EW // C
UNROLL = 5
NSUPER = NCHUNK // UNROLL


def _leaky(x):
    return jnp.where(x >= 0, x, 0.01 * x)


def _node_proj_body(atom_ref, w_ref, out_ref):
    out_ref[...] = jnp.dot(atom_ref[...], w_ref[...],
                           preferred_element_type=jnp.float32,
                           precision=lax.Precision.HIGHEST)


def _node_proj(atom_feats, w1b):
    return pl.pallas_call(
        _node_proj_body,
        out_shape=jax.ShapeDtypeStruct((N_NODES, OUT_DIM), jnp.float32),
    )(atom_feats, w1b)


def _pack_bf16_pairs(aw):
    lo = lax.bitcast_convert_type(aw[:, :PK].astype(jnp.bfloat16), jnp.uint16)
    hi = lax.bitcast_convert_type(aw[:, PK:].astype(jnp.bfloat16), jnp.uint16)
    packed = (hi.astype(jnp.uint32) << 16) | lo.astype(jnp.uint32)
    return lax.bitcast_convert_type(packed, jnp.int32)


_MHI = np.uint32(0xFFFF0000)


def _unpack2(w):
    u = lax.bitcast_convert_type(w, jnp.uint32)
    lo = lax.bitcast_convert_type(u << 16, jnp.float32)
    hi = lax.bitcast_convert_type(u & _MHI, jnp.float32)
    return lo, hi


@functools.partial(
    pl.kernel,
    mesh=plsc.VectorSubcoreMesh(core_axis_name="c", subcore_axis_name="s"),
    out_type=jax.ShapeDtypeStruct((N_EDGES, OUT_DIM), jnp.float32),
    scratch_types=[
        pltpu.VMEM((EW,), jnp.int32),
        pltpu.VMEM((EW,), jnp.int32),
        pltpu.VMEM((2 * UNROLL, C, PK), jnp.int32),
        pltpu.VMEM((UNROLL, C, OUT_DIM), jnp.float32),
        pltpu.SemaphoreType.DMA,
        pltpu.SemaphoreType.DMA,
    ],
    compiler_params=pltpu.CompilerParams(use_tc_tiling_on_sc=False),
)
def _gather_add(table, src_hbm, dst_hbm, out_hbm,
                idx_s, idx_d, rows, srows, sem_g, sem_w):
    wid = lax.axis_index("s") * NC + lax.axis_index("c")
    base = wid * EW
    pltpu.sync_copy(src_hbm.at[pl.ds(base, EW)], idx_s)
    pltpu.sync_copy(dst_hbm.at[pl.ds(base, EW)], idx_d)

    def super_chunk(t, carry):
        j0 = t * UNROLL
        gathers = []
        for u in range(UNROLL):
            sl = pl.ds((j0 + u) * C, C)
            c1 = pltpu.async_copy(table.at[idx_s.at[sl]], rows.at[2 * u], sem_g)
            c2 = pltpu.async_copy(table.at[idx_d.at[sl]], rows.at[2 * u + 1],
                                  sem_g)
            gathers.append((c1, c2))
        writebacks = []
        for u in range(UNROLL):
            c1, c2 = gathers[u]
            c1.wait()
            c2.wait()

            def add_row(e, carry2, u=u):
                for k in range(PK // 16):
                    sl2 = pl.ds(k * 16, 16)
                    al, ah = _unpack2(rows[2 * u, e, sl2])
                    bl, bh = _unpack2(rows[2 * u + 1, e, sl2])
                    srows[u, e, pl.ds(k * 16, 16)] = al + bl
                    srows[u, e, pl.ds(PK + k * 16, 16)] = ah + bh
                return carry2

            lax.fori_loop(0, C, add_row, 0)
            off = base + (j0 + u) * C
            wb = pltpu.async_copy(srows.at[u], out_hbm.at[pl.ds(off, C)],
                                  sem_w)
            writebacks.append(wb)
        for wb in writebacks:
            wb.wait()
        return carry

    lax.fori_loop(0, NSUPER, super_chunk, 0)


BE = 2000


def _mlp_body(bond_ref, s_ref, w1a_ref, b1_ref, w2_ref, b2_ref,
              w3_ref, b3_ref, out_ref):
    h = jnp.dot(bond_ref[...], w1a_ref[...],
                preferred_element_type=jnp.float32)
    h = _leaky(h + s_ref[...] + b1_ref[...])
    h = _leaky(jnp.dot(h.astype(jnp.bfloat16), w2_ref[...],
                       preferred_element_type=jnp.float32) + b2_ref[...])
    h = _leaky(jnp.dot(h.astype(jnp.bfloat16), w3_ref[...],
                       preferred_element_type=jnp.float32) + b3_ref[...])
    out_ref[...] = h


def _edge_mlp(bond_feats, s, w1a, b1, w2, b2, w3, b3):
    grid = (N_EDGES // BE,)
    bond_feats = bond_feats.astype(jnp.bfloat16)
    w1a = w1a.astype(jnp.bfloat16)
    w2 = w2.astype(jnp.bfloat16)
    w3 = w3.astype(jnp.bfloat16)
    full = lambda shape: pl.BlockSpec(shape, lambda i: (0, 0))
    return pl.pallas_call(
        _mlp_body,
        grid=grid,
        in_specs=[
            pl.BlockSpec((BE, D_EDGE), lambda i: (i, 0)),
            pl.BlockSpec((BE, OUT_DIM), lambda i: (i, 0)),
            full((D_EDGE, OUT_DIM)),
            full((1, OUT_DIM)),
            full((OUT_DIM, OUT_DIM)),
            full((1, OUT_DIM)),
            full((OUT_DIM, OUT_DIM)),
            full((1, OUT_DIM)),
        ],
        out_specs=pl.BlockSpec((BE, OUT_DIM), lambda i: (i, 0)),
        out_shape=jax.ShapeDtypeStruct((N_EDGES, OUT_DIM), jnp.float32),
    )(bond_feats, s, w1a, b1, w2, b2, w3, b3)


def kernel(atom_feats, bond_feats, edge_index, W1, b1, W2, b2, W3, b3):
    w1a = W1[:D_EDGE]
    w1b = W1[D_EDGE:]
    src = edge_index[0]
    dst = edge_index[1]
    atom_w = _pack_bf16_pairs(_node_proj(atom_feats, w1b))
    s = _gather_add(atom_w, src, dst)
    return _edge_mlp(bond_feats, s,
                     w1a, b1.reshape(1, -1),
                     W2, b2.reshape(1, -1),
                     W3, b3.reshape(1, -1))


# speedup vs baseline: 3.0654x; 1.0052x over previous
import functools

import jax
import jax.numpy as jnp
import numpy as np
from jax import lax
from jax.experimental import pallas as pl
from jax.experimental.pallas import tpu as pltpu
from jax.experimental.pallas import tpu_sc as plsc

N_NODES = 10000
N_EDGES = 320000
D_FEAT = 128
D_EDGE = 16
OUT_DIM = 128
PK = OUT_DIM // 2

NC = 2
NS = 16
NW = NC * NS
EW = N_EDGES // NW
C = 80
NCHUNK = EW // C
UNROLL = 5
NSUPER = NCHUNK // UNROLL


def _leaky(x):
    return jnp.where(x >= 0, x, 0.01 * x)


def _node_proj_body(atom_ref, w_ref, out_ref):
    out_ref[...] = jnp.dot(atom_ref[...], w_ref[...],
                           preferred_element_type=jnp.float32,
                           precision=lax.Precision.HIGHEST)


def _node_proj(atom_feats, w1b):
    return pl.pallas_call(
        _node_proj_body,
        out_shape=jax.ShapeDtypeStruct((N_NODES, OUT_DIM), jnp.float32),
    )(atom_feats, w1b)


def _pack_bf16_pairs(aw):
    lo = lax.bitcast_convert_type(aw[:, :PK].astype(jnp.bfloat16), jnp.uint16)
    hi = lax.bitcast_convert_type(aw[:, PK:].astype(jnp.bfloat16), jnp.uint16)
    packed = (hi.astype(jnp.uint32) << 16) | lo.astype(jnp.uint32)
    return lax.bitcast_convert_type(packed, jnp.int32)


_MHI = np.uint32(0xFFFF0000)


def _unpack2(w):
    u = lax.bitcast_convert_type(w, jnp.uint32)
    lo = lax.bitcast_convert_type(u << 16, jnp.float32)
    hi = lax.bitcast_convert_type(u & _MHI, jnp.float32)
    return lo, hi


@functools.partial(
    pl.kernel,
    mesh=plsc.VectorSubcoreMesh(core_axis_name="c", subcore_axis_name="s"),
    out_type=jax.ShapeDtypeStruct((N_EDGES, OUT_DIM), jnp.float32),
    scratch_types=[
        pltpu.VMEM((EW,), jnp.int32),
        pltpu.VMEM((EW,), jnp.int32),
        pltpu.VMEM((2 * UNROLL, C, PK), jnp.int32),
        pltpu.VMEM((UNROLL, C, OUT_DIM), jnp.float32),
        pltpu.SemaphoreType.DMA,
        pltpu.SemaphoreType.DMA,
    ],
    compiler_params=pltpu.CompilerParams(use_tc_tiling_on_sc=False),
)
def _gather_add(table, src_hbm, dst_hbm, out_hbm,
                idx_s, idx_d, rows, srows, sem_g, sem_w):
    wid = lax.axis_index("s") * NC + lax.axis_index("c")
    base = wid * EW
    pltpu.sync_copy(src_hbm.at[pl.ds(base, EW)], idx_s)
    pltpu.sync_copy(dst_hbm.at[pl.ds(base, EW)], idx_d)

    def super_chunk(t, carry):
        j0 = t * UNROLL
        gathers = []
        for u in range(UNROLL):
            sl = pl.ds((j0 + u) * C, C)
            c1 = pltpu.async_copy(table.at[idx_s.at[sl]], rows.at[2 * u], sem_g)
            c2 = pltpu.async_copy(table.at[idx_d.at[sl]], rows.at[2 * u + 1],
                                  sem_g)
            gathers.append((c1, c2))
        writebacks = []
        for u in range(UNROLL):
            c1, c2 = gathers[u]
            c1.wait()
            c2.wait()

            def add_row(e, carry2, u=u):
                for k in range(PK // 16):
                    sl2 = pl.ds(k * 16, 16)
                    al, ah = _unpack2(rows[2 * u, e, sl2])
                    bl, bh = _unpack2(rows[2 * u + 1, e, sl2])
                    srows[u, e, pl.ds(k * 16, 16)] = al + bl
                    srows[u, e, pl.ds(PK + k * 16, 16)] = ah + bh
                return carry2

            lax.fori_loop(0, C, add_row, 0, unroll=4)
            off = base + (j0 + u) * C
            wb = pltpu.async_copy(srows.at[u], out_hbm.at[pl.ds(off, C)],
                                  sem_w)
            writebacks.append(wb)
        for wb in writebacks:
            wb.wait()
        return carry

    lax.fori_loop(0, NSUPER, super_chunk, 0)


BE = 2000


def _mlp_body(bond_ref, s_ref, w1a_ref, b1_ref, w2_ref, b2_ref,
              w3_ref, b3_ref, out_ref):
    h = jnp.dot(bond_ref[...], w1a_ref[...],
                preferred_element_type=jnp.float32)
    h = _leaky(h + s_ref[...] + b1_ref[...])
    h = _leaky(jnp.dot(h.astype(jnp.bfloat16), w2_ref[...],
                       preferred_element_type=jnp.float32) + b2_ref[...])
    h = _leaky(jnp.dot(h.astype(jnp.bfloat16), w3_ref[...],
                       preferred_element_type=jnp.float32) + b3_ref[...])
    out_ref[...] = h


def _edge_mlp(bond_feats, s, w1a, b1, w2, b2, w3, b3):
    grid = (N_EDGES // BE,)
    bond_feats = bond_feats.astype(jnp.bfloat16)
    w1a = w1a.astype(jnp.bfloat16)
    w2 = w2.astype(jnp.bfloat16)
    w3 = w3.astype(jnp.bfloat16)
    full = lambda shape: pl.BlockSpec(shape, lambda i: (0, 0))
    return pl.pallas_call(
        _mlp_body,
        grid=grid,
        in_specs=[
            pl.BlockSpec((BE, D_EDGE), lambda i: (i, 0)),
            pl.BlockSpec((BE, OUT_DIM), lambda i: (i, 0)),
            full((D_EDGE, OUT_DIM)),
            full((1, OUT_DIM)),
            full((OUT_DIM, OUT_DIM)),
            full((1, OUT_DIM)),
            full((OUT_DIM, OUT_DIM)),
            full((1, OUT_DIM)),
        ],
        out_specs=pl.BlockSpec((BE, OUT_DIM), lambda i: (i, 0)),
        out_shape=jax.ShapeDtypeStruct((N_EDGES, OUT_DIM), jnp.float32),
    )(bond_feats, s, w1a, b1, w2, b2, w3, b3)


def kernel(atom_feats, bond_feats, edge_index, W1, b1, W2, b2, W3, b3):
    w1a = W1[:D_EDGE]
    w1b = W1[D_EDGE:]
    src = edge_index[0]
    dst = edge_index[1]
    atom_w = _pack_bf16_pairs(_node_proj(atom_feats, w1b))
    s = _gather_add(atom_w, src, dst)
    return _edge_mlp(bond_feats, s,
                     w1a, b1.reshape(1, -1),
                     W2, b2.reshape(1, -1),
                     W3, b3.reshape(1, -1))
